# baseline (speedup 1.0000x reference)
import jax
import jax.numpy as jnp
from jax.experimental import pallas as pl
from jax.experimental.pallas import tpu as pltpu

_BM_CAST = 384
_BM = 1000


def _mm_cast_kernel(a_ref, x_ref, o_ref, abf_ref):
    abf = a_ref[...].astype(jnp.bfloat16)
    abf_ref[...] = abf
    o_ref[...] = jnp.dot(abf, x_ref[...].astype(jnp.bfloat16),
                         preferred_element_type=jnp.float32)


def _mm_cast(a, xmat):
    n = a.shape[0]
    w = xmat.shape[1]
    return pl.pallas_call(
        _mm_cast_kernel,
        grid=(pl.cdiv(n, _BM_CAST),),
        in_specs=[
            pl.BlockSpec((_BM_CAST, n), lambda i: (i, 0)),
            pl.BlockSpec((n, w), lambda i: (0, 0)),
        ],
        out_specs=[
            pl.BlockSpec((_BM_CAST, w), lambda i: (i, 0)),
            pl.BlockSpec((_BM_CAST, n), lambda i: (i, 0)),
        ],
        out_shape=[
            jax.ShapeDtypeStruct((n, w), jnp.float32),
            jax.ShapeDtypeStruct((n, n), jnp.bfloat16),
        ],
        compiler_params=pltpu.CompilerParams(
            dimension_semantics=("arbitrary",),
        ),
    )(a, xmat)


def _mm_bf_kernel(a_ref, x_ref, o_ref):
    o_ref[...] = jnp.dot(a_ref[...], x_ref[...].astype(jnp.bfloat16),
                         preferred_element_type=jnp.float32)


def _mm_bf(a, xmat):
    n = a.shape[0]
    w = xmat.shape[1]
    return pl.pallas_call(
        _mm_bf_kernel,
        grid=(pl.cdiv(n, _BM),),
        in_specs=[
            pl.BlockSpec((_BM, n), lambda i: (i, 0)),
            pl.BlockSpec((n, w), lambda i: (0, 0)),
        ],
        out_specs=pl.BlockSpec((_BM, w), lambda i: (i, 0)),
        out_shape=jax.ShapeDtypeStruct((n, w), jnp.float32),
        compiler_params=pltpu.CompilerParams(
            dimension_semantics=("arbitrary",),
        ),
    )(a, xmat)


def _proj_kernel(x_ref, w_ref, o_ref):
    o_ref[...] = jax.lax.dot_general(
        x_ref[...], w_ref[...],
        dimension_numbers=(((1,), (1,)), ((), ())),
        preferred_element_type=jnp.float32)


def _proj(x, wcat):
    n, d = x.shape
    h = wcat.shape[0]
    return pl.pallas_call(
        _proj_kernel,
        grid=(pl.cdiv(n, _BM),),
        in_specs=[
            pl.BlockSpec((_BM, d), lambda i: (i, 0)),
            pl.BlockSpec((h, d), lambda i: (0, 0)),
        ],
        out_specs=pl.BlockSpec((_BM, h), lambda i: (i, 0)),
        out_shape=jax.ShapeDtypeStruct((n, h), jnp.float32),
    )(x, wcat)


def _combine_kernel(g1_ref, g2_ref, g3_ref, s1_ref, s2_ref, s4_ref, s8_ref,
                    b_ref, o_ref):
    b = b_ref[...]
    o_ref[:, 0:16] = jnp.maximum(g1_ref[:, 0:16] + b[:, 0:16], 0.0)
    o_ref[:, 16:32] = jnp.maximum(g2_ref[:, 0:16] + b[:, 16:32], 0.0)
    o_ref[:, 32:48] = jnp.maximum(g3_ref[...] + b[:, 32:48], 0.0)
    o_ref[:, 48:64] = jnp.maximum(
        s1_ref[:, 0:16] - s2_ref[:, 0:16] + b[:, 48:64], 0.0)
    o_ref[:, 64:80] = jnp.maximum(
        s2_ref[:, 16:32] - s4_ref[:, 0:16] + b[:, 64:80], 0.0)
    o_ref[:, 80:96] = jnp.maximum(
        s4_ref[:, 16:32] - s8_ref[...] + b[:, 80:96], 0.0)


def _combine(g1, g2, g3, s1, s2, s4, s8, bcat):
    n = g1.shape[0]
    args = (g1, g2, g3, s1, s2, s4, s8)
    in_specs = [pl.BlockSpec((_BM, a.shape[1]), lambda i: (i, 0))
                for a in args]
    in_specs.append(pl.BlockSpec((1, 96), lambda i: (0, 0)))
    return pl.pallas_call(
        _combine_kernel,
        grid=(pl.cdiv(n, _BM),),
        in_specs=in_specs,
        out_specs=pl.BlockSpec((_BM, 96), lambda i: (i, 0)),
        out_shape=jax.ShapeDtypeStruct((n, 96), jnp.float32),
    )(*args, bcat)


def kernel(x, gcn_mat, sct_mat, W0, W1, W2, W3, W4, W5,
           b0, b1, b2, b3, b4, b5):
    wcat = jnp.concatenate([W0, W1, W2, W3, W4, W5], axis=0)
    bcat = jnp.concatenate([b0, b1, b2, b3, b4, b5], axis=1)

    h = _proj(x, wcat)

    g1, gcn_bf = _mm_cast(gcn_mat, h[:, 0:48])
    g2 = _mm_bf(gcn_bf, g1[:, 16:48])
    g3 = _mm_bf(gcn_bf, g2[:, 16:32])

    s1, sct_bf = _mm_cast(sct_mat, h[:, 48:96])
    s2 = _mm_bf(sct_bf, s1)
    s3 = _mm_bf(sct_bf, s2[:, 16:48])
    s4 = _mm_bf(sct_bf, s3)
    s5 = _mm_bf(sct_bf, s4[:, 16:32])
    s6 = _mm_bf(sct_bf, s5)
    s7 = _mm_bf(sct_bf, s6)
    s8 = _mm_bf(sct_bf, s7)

    return _combine(g1, g2, g3, s1, s2, s4, s8, bcat)


# speedup vs baseline: 1.0055x; 1.0055x over previous
import jax
import jax.numpy as jnp
from jax.experimental import pallas as pl
from jax.experimental.pallas import tpu as pltpu

_BM_CAST = 384
_BM = 1000


def _mm_cast_kernel(a_ref, x_ref, o_ref, abf_ref):
    abf = a_ref[...].astype(jnp.bfloat16)
    abf_ref[...] = abf
    o_ref[...] = jnp.dot(abf, x_ref[...].astype(jnp.bfloat16),
                         preferred_element_type=jnp.float32)


def _mm_cast(a, xmat):
    n = a.shape[0]
    w = xmat.shape[1]
    return pl.pallas_call(
        _mm_cast_kernel,
        grid=(pl.cdiv(n, _BM_CAST),),
        in_specs=[
            pl.BlockSpec((_BM_CAST, n), lambda i: (i, 0)),
            pl.BlockSpec((n, w), lambda i: (0, 0)),
        ],
        out_specs=[
            pl.BlockSpec((_BM_CAST, w), lambda i: (i, 0)),
            pl.BlockSpec((_BM_CAST, n), lambda i: (i, 0)),
        ],
        out_shape=[
            jax.ShapeDtypeStruct((n, w), jnp.float32),
            jax.ShapeDtypeStruct((n, n), jnp.bfloat16),
        ],
        compiler_params=pltpu.CompilerParams(
            dimension_semantics=("arbitrary",),
        ),
    )(a, xmat)


def _mm_bf_kernel(a_ref, x_ref, o_ref):
    o_ref[...] = jnp.dot(a_ref[...], x_ref[...].astype(jnp.bfloat16),
                         preferred_element_type=jnp.float32)


def _mm_bf(a, xmat):
    n = a.shape[0]
    w = xmat.shape[1]
    return pl.pallas_call(
        _mm_bf_kernel,
        grid=(pl.cdiv(n, _BM),),
        in_specs=[
            pl.BlockSpec((_BM, n), lambda i: (i, 0)),
            pl.BlockSpec((n, w), lambda i: (0, 0)),
        ],
        out_specs=pl.BlockSpec((_BM, w), lambda i: (i, 0)),
        out_shape=jax.ShapeDtypeStruct((n, w), jnp.float32),
        compiler_params=pltpu.CompilerParams(
            dimension_semantics=("arbitrary",),
        ),
    )(a, xmat)


def _proj_kernel(x_ref, w_ref, o_ref):
    o_ref[...] = jax.lax.dot_general(
        x_ref[...], w_ref[...],
        dimension_numbers=(((1,), (1,)), ((), ())),
        preferred_element_type=jnp.float32)


def _proj(x, wcat):
    n, d = x.shape
    h = wcat.shape[0]
    return pl.pallas_call(
        _proj_kernel,
        grid=(pl.cdiv(n, _BM),),
        in_specs=[
            pl.BlockSpec((_BM, d), lambda i: (i, 0)),
            pl.BlockSpec((h, d), lambda i: (0, 0)),
        ],
        out_specs=pl.BlockSpec((_BM, h), lambda i: (i, 0)),
        out_shape=jax.ShapeDtypeStruct((n, h), jnp.float32),
    )(x, wcat)


def _last_combine_kernel(a_ref, x_ref, g1_ref, g2_ref, g3_ref, s1_ref,
                         s2_ref, s4_ref, b_ref, o_ref):
    s8 = jnp.dot(a_ref[...], x_ref[...].astype(jnp.bfloat16),
                 preferred_element_type=jnp.float32)
    b = b_ref[...]
    o_ref[:, 0:16] = jnp.maximum(g1_ref[:, 0:16] + b[:, 0:16], 0.0)
    o_ref[:, 16:32] = jnp.maximum(g2_ref[:, 0:16] + b[:, 16:32], 0.0)
    o_ref[:, 32:48] = jnp.maximum(g3_ref[...] + b[:, 32:48], 0.0)
    o_ref[:, 48:64] = jnp.maximum(
        s1_ref[:, 0:16] - s2_ref[:, 0:16] + b[:, 48:64], 0.0)
    o_ref[:, 64:80] = jnp.maximum(
        s2_ref[:, 16:32] - s4_ref[:, 0:16] + b[:, 64:80], 0.0)
    o_ref[:, 80:96] = jnp.maximum(
        s4_ref[:, 16:32] - s8 + b[:, 80:96], 0.0)


def _last_combine(a, xmat, g1, g2, g3, s1, s2, s4, bcat):
    n = a.shape[0]
    small = (g1, g2, g3, s1, s2, s4)
    in_specs = [
        pl.BlockSpec((_BM, n), lambda i: (i, 0)),
        pl.BlockSpec((n, xmat.shape[1]), lambda i: (0, 0)),
    ]
    in_specs += [pl.BlockSpec((_BM, m.shape[1]), lambda i: (i, 0))
                 for m in small]
    in_specs.append(pl.BlockSpec((1, 96), lambda i: (0, 0)))
    return pl.pallas_call(
        _last_combine_kernel,
        grid=(pl.cdiv(n, _BM),),
        in_specs=in_specs,
        out_specs=pl.BlockSpec((_BM, 96), lambda i: (i, 0)),
        out_shape=jax.ShapeDtypeStruct((n, 96), jnp.float32),
        compiler_params=pltpu.CompilerParams(
            dimension_semantics=("arbitrary",),
        ),
    )(a, xmat, *small, bcat)


def kernel(x, gcn_mat, sct_mat, W0, W1, W2, W3, W4, W5,
           b0, b1, b2, b3, b4, b5):
    wcat = jnp.concatenate([W0, W1, W2, W3, W4, W5], axis=0)
    bcat = jnp.concatenate([b0, b1, b2, b3, b4, b5], axis=1)

    h = _proj(x, wcat)

    g1, gcn_bf = _mm_cast(gcn_mat, h[:, 0:48])
    g2 = _mm_bf(gcn_bf, g1[:, 16:48])
    g3 = _mm_bf(gcn_bf, g2[:, 16:32])

    s1, sct_bf = _mm_cast(sct_mat, h[:, 48:96])
    s2 = _mm_bf(sct_bf, s1)
    s3 = _mm_bf(sct_bf, s2[:, 16:48])
    s4 = _mm_bf(sct_bf, s3)
    s5 = _mm_bf(sct_bf, s4[:, 16:32])
    s6 = _mm_bf(sct_bf, s5)
    s7 = _mm_bf(sct_bf, s6)

    return _last_combine(sct_bf, s7, g1, g2, g3, s1, s2, s4, bcat)


# speedup vs baseline: 1.0150x; 1.0094x over previous
import jax
import jax.numpy as jnp
from jax.experimental import pallas as pl
from jax.experimental.pallas import tpu as pltpu

_BMC = 400
_BMB = 1000

_HBM = pltpu.MemorySpace.HBM
_VMEM = pltpu.MemorySpace.VMEM


def _mega_body(x, gcn, sct, wcat_ref, bcat_ref, v, gbf, sbf, xbuf, sem):
    n = gcn.shape[0]
    bmc = _BMC if n % _BMC == 0 else (200 if n % 200 == 0 else n)
    bmb = _BMB if n % _BMB == 0 else n
    bf = jnp.bfloat16
    f32 = jnp.float32

    def dot(a, b):
        return jnp.dot(a, b, preferred_element_type=f32)

    def load_group(g):
        cp = pltpu.make_async_copy(v.at[:, 128 * g:128 * (g + 1)], xbuf, sem)
        cp.start()
        cp.wait()

    def pad128(y):
        w = y.shape[1]
        if w == 128:
            return y
        return jnp.concatenate(
            [y, jnp.zeros((y.shape[0], 128 - w), f32)], axis=1)

    def proj_body(x_ref, h_ref):
        hv = jax.lax.dot_general(
            x_ref[...], wcat_ref[...],
            dimension_numbers=(((1,), (1,)), ((), ())),
            preferred_element_type=f32)
        h_ref[...] = pad128(hv)

    pltpu.emit_pipeline(
        proj_body, grid=(n // bmb,),
        in_specs=[pl.BlockSpec((bmb, 128), lambda i: (i, 0))],
        out_specs=[pl.BlockSpec((bmb, 128), lambda i: (i, 0))],
    )(x, v)

    def prop_pass(a, c0, c1, ygroup, bm, cache=None):
        def body(*refs):
            a_ref = refs[0]
            y_ref = refs[1]
            av = a_ref[...]
            if av.dtype == f32:
                abf = av.astype(bf)
                refs[2][...] = abf
            else:
                abf = av
            y = dot(abf, xbuf[:, c0:c1].astype(bf))
            y_ref[...] = pad128(y)

        in_specs = [pl.BlockSpec((bm, n), lambda i: (i, 0))]
        out_specs = [pl.BlockSpec((bm, 128), lambda i, g=ygroup: (i, g))]
        refs = [a, v]
        if cache is not None:
            out_specs.append(pl.BlockSpec((bm, n), lambda i: (i, 0)))
            refs.append(cache)
        pltpu.emit_pipeline(
            body, grid=(n // bm,),
            in_specs=in_specs, out_specs=out_specs,
        )(*refs)

    load_group(0)
    prop_pass(gcn, 0, 48, 1, bmc, cache=gbf)
    prop_pass(sct, 48, 96, 4, bmc, cache=sbf)
    load_group(1)
    prop_pass(gbf, 16, 48, 2, bmb)
    load_group(2)
    prop_pass(gbf, 16, 32, 3, bmb)
    load_group(4)
    prop_pass(sbf, 0, 48, 5, bmb)
    load_group(5)
    prop_pass(sbf, 16, 48, 6, bmb)
    load_group(6)
    prop_pass(sbf, 0, 32, 7, bmb)
    load_group(7)
    prop_pass(sbf, 16, 32, 8, bmb)
    load_group(8)
    prop_pass(sbf, 0, 16, 9, bmb)
    load_group(9)
    prop_pass(sbf, 0, 16, 10, bmb)
    load_group(10)
    prop_pass(sbf, 0, 16, 11, bmb)


def _combine_kernel(g1_ref, g2_ref, g3_ref, s1_ref, s2_ref, s4_ref, s8_ref,
                    b_ref, o_ref):
    b = b_ref[...]
    o_ref[:, 0:16] = jnp.maximum(g1_ref[:, 0:16] + b[:, 0:16], 0.0)
    o_ref[:, 16:32] = jnp.maximum(g2_ref[:, 0:16] + b[:, 16:32], 0.0)
    o_ref[:, 32:48] = jnp.maximum(g3_ref[:, 0:16] + b[:, 32:48], 0.0)
    o_ref[:, 48:64] = jnp.maximum(
        s1_ref[:, 0:16] - s2_ref[:, 0:16] + b[:, 48:64], 0.0)
    o_ref[:, 64:80] = jnp.maximum(
        s2_ref[:, 16:32] - s4_ref[:, 0:16] + b[:, 64:80], 0.0)
    o_ref[:, 80:96] = jnp.maximum(
        s4_ref[:, 16:32] - s8_ref[:, 0:16] + b[:, 80:96], 0.0)


def _combine(v, bcat):
    n = v.shape[0]
    bm = 1000 if n % 1000 == 0 else n
    groups = (1, 2, 3, 4, 5, 7, 11)
    in_specs = [pl.BlockSpec((bm, 128), lambda i, g=g: (i, g))
                for g in groups]
    in_specs.append(pl.BlockSpec((1, 96), lambda i: (0, 0)))
    return pl.pallas_call(
        _combine_kernel,
        grid=(n // bm,),
        in_specs=in_specs,
        out_specs=pl.BlockSpec((bm, 96), lambda i: (i, 0)),
        out_shape=jax.ShapeDtypeStruct((n, 96), jnp.float32),
    )(*([v] * len(groups)), bcat)


def kernel(x, gcn_mat, sct_mat, W0, W1, W2, W3, W4, W5,
           b0, b1, b2, b3, b4, b5):
    n = x.shape[0]
    wcat = jnp.concatenate([W0, W1, W2, W3, W4, W5], axis=0)
    bcat = jnp.concatenate([b0, b1, b2, b3, b4, b5], axis=1)

    v, _, _ = pl.pallas_call(
        _mega_body,
        in_specs=[
            pl.BlockSpec(memory_space=_HBM),
            pl.BlockSpec(memory_space=_HBM),
            pl.BlockSpec(memory_space=_HBM),
            pl.BlockSpec(memory_space=_VMEM),
            pl.BlockSpec(memory_space=_VMEM),
        ],
        out_specs=[pl.BlockSpec(memory_space=_HBM)] * 3,
        out_shape=[
            jax.ShapeDtypeStruct((n, 12 * 128), jnp.float32),
            jax.ShapeDtypeStruct((n, n), jnp.bfloat16),
            jax.ShapeDtypeStruct((n, n), jnp.bfloat16),
        ],
        scratch_shapes=[
            pltpu.VMEM((n, 128), jnp.float32),
            pltpu.SemaphoreType.DMA,
        ],
        compiler_params=pltpu.CompilerParams(
            vmem_limit_bytes=100 * 1024 * 1024,
        ),
    )(x, gcn_mat, sct_mat, wcat, bcat)
    return _combine(v, bcat)


# speedup vs baseline: 1.0534x; 1.0378x over previous
import jax
import jax.numpy as jnp
from jax.experimental import pallas as pl
from jax.experimental.pallas import tpu as pltpu

_BMC = 400
_BMB = 1000

_HBM = pltpu.MemorySpace.HBM
_VMEM = pltpu.MemorySpace.VMEM


def _mega_body(x, gcn, sct, wcat_ref, bcat_ref, v, gbf, sbf, x0, x1, x2):
    n = gcn.shape[0]
    bmc = _BMC if n % _BMC == 0 else (200 if n % 200 == 0 else n)
    bmb = _BMB if n % _BMB == 0 else n
    bf = jnp.bfloat16
    f32 = jnp.float32

    def dot(a, b):
        return jnp.dot(a, b, preferred_element_type=f32)

    def pad128(y):
        w = y.shape[1]
        if w == 128:
            return y
        return jnp.concatenate(
            [y, jnp.zeros((y.shape[0], 128 - w), f32)], axis=1)

    def proj_body(x_ref, h_ref):
        hv = jax.lax.dot_general(
            x_ref[...], wcat_ref[...],
            dimension_numbers=(((1,), (1,)), ((), ())),
            preferred_element_type=f32)
        h_ref[...] = pad128(hv)

    pltpu.emit_pipeline(
        proj_body, grid=(n // bmb,),
        in_specs=[pl.BlockSpec((bmb, 128), lambda i: (i, 0))],
        out_specs=[pl.BlockSpec((bmb, 128), lambda i: (i, 0))],
    )(x, x0)

    def prop_pass(a, xin, c0, c1, bm, xout=None, group=None, cache=None):
        def body(*refs):
            a_ref = refs[0]
            av = a_ref[...]
            if av.dtype == f32:
                abf = av.astype(bf)
                refs[-1][...] = abf
            else:
                abf = av
            y = pad128(dot(abf, xin[:, c0:c1].astype(bf)))
            k = 1
            if xout is not None:
                refs[k][...] = y
                k += 1
            if group is not None:
                refs[k][...] = y

        in_specs = [pl.BlockSpec((bm, n), lambda i: (i, 0))]
        out_specs = []
        refs = [a]
        if xout is not None:
            out_specs.append(pl.BlockSpec((bm, 128), lambda i: (i, 0)))
            refs.append(xout)
        if group is not None:
            out_specs.append(pl.BlockSpec((bm, 128), lambda i, g=group: (i, g)))
            refs.append(v)
        if cache is not None:
            out_specs.append(pl.BlockSpec((bm, n), lambda i: (i, 0)))
            refs.append(cache)
        pltpu.emit_pipeline(
            body, grid=(n // bm,),
            in_specs=in_specs, out_specs=out_specs,
        )(*refs)

    prop_pass(gcn, x0, 0, 48, bmc, xout=x1, group=0, cache=gbf)
    prop_pass(sct, x0, 48, 96, bmc, xout=x2, group=3, cache=sbf)
    prop_pass(gbf, x1, 16, 48, bmb, xout=x0, group=1)
    prop_pass(gbf, x0, 16, 32, bmb, group=2)
    prop_pass(sbf, x2, 0, 48, bmb, xout=x1, group=4)
    prop_pass(sbf, x1, 16, 48, bmb, xout=x2)
    prop_pass(sbf, x2, 0, 32, bmb, xout=x1, group=5)
    prop_pass(sbf, x1, 16, 32, bmb, xout=x2)
    prop_pass(sbf, x2, 0, 16, bmb, xout=x1)
    prop_pass(sbf, x1, 0, 16, bmb, xout=x2)
    prop_pass(sbf, x2, 0, 16, bmb, group=6)


def _combine_kernel(g1_ref, g2_ref, g3_ref, s1_ref, s2_ref, s4_ref, s8_ref,
                    b_ref, o_ref):
    b = b_ref[...]
    o_ref[:, 0:16] = jnp.maximum(g1_ref[:, 0:16] + b[:, 0:16], 0.0)
    o_ref[:, 16:32] = jnp.maximum(g2_ref[:, 0:16] + b[:, 16:32], 0.0)
    o_ref[:, 32:48] = jnp.maximum(g3_ref[:, 0:16] + b[:, 32:48], 0.0)
    o_ref[:, 48:64] = jnp.maximum(
        s1_ref[:, 0:16] - s2_ref[:, 0:16] + b[:, 48:64], 0.0)
    o_ref[:, 64:80] = jnp.maximum(
        s2_ref[:, 16:32] - s4_ref[:, 0:16] + b[:, 64:80], 0.0)
    o_ref[:, 80:96] = jnp.maximum(
        s4_ref[:, 16:32] - s8_ref[:, 0:16] + b[:, 80:96], 0.0)


def _combine(v, bcat):
    n = v.shape[0]
    bm = 1000 if n % 1000 == 0 else n
    groups = (0, 1, 2, 3, 4, 5, 6)
    in_specs = [pl.BlockSpec((bm, 128), lambda i, g=g: (i, g))
                for g in groups]
    in_specs.append(pl.BlockSpec((1, 96), lambda i: (0, 0)))
    return pl.pallas_call(
        _combine_kernel,
        grid=(n // bm,),
        in_specs=in_specs,
        out_specs=pl.BlockSpec((bm, 96), lambda i: (i, 0)),
        out_shape=jax.ShapeDtypeStruct((n, 96), jnp.float32),
    )(*([v] * len(groups)), bcat)


def kernel(x, gcn_mat, sct_mat, W0, W1, W2, W3, W4, W5,
           b0, b1, b2, b3, b4, b5):
    n = x.shape[0]
    wcat = jnp.concatenate([W0, W1, W2, W3, W4, W5], axis=0)
    bcat = jnp.concatenate([b0, b1, b2, b3, b4, b5], axis=1)

    v, _, _ = pl.pallas_call(
        _mega_body,
        in_specs=[
            pl.BlockSpec(memory_space=_HBM),
            pl.BlockSpec(memory_space=_HBM),
            pl.BlockSpec(memory_space=_HBM),
            pl.BlockSpec(memory_space=_VMEM),
            pl.BlockSpec(memory_space=_VMEM),
        ],
        out_specs=[pl.BlockSpec(memory_space=_HBM)] * 3,
        out_shape=[
            jax.ShapeDtypeStruct((n, 7 * 128), jnp.float32),
            jax.ShapeDtypeStruct((n, n), jnp.bfloat16),
            jax.ShapeDtypeStruct((n, n), jnp.bfloat16),
        ],
        scratch_shapes=[
            pltpu.VMEM((n, 128), jnp.float32),
            pltpu.VMEM((n, 128), jnp.float32),
            pltpu.VMEM((n, 128), jnp.float32),
        ],
        compiler_params=pltpu.CompilerParams(
            vmem_limit_bytes=100 * 1024 * 1024,
        ),
    )(x, gcn_mat, sct_mat, wcat, bcat)
    return _combine(v, bcat)


# speedup vs baseline: 1.0649x; 1.0109x over previous
import jax
import jax.numpy as jnp
from jax.experimental import pallas as pl
from jax.experimental.pallas import tpu as pltpu

_BMC = 400
_BMB = 1000

_HBM = pltpu.MemorySpace.HBM
_VMEM = pltpu.MemorySpace.VMEM


def _mega_body(x, gcn, sct, wcat_ref, bcat_ref, v, gbf, sbf, x0, x1, x2):
    n = gcn.shape[0]
    bmc = _BMC if n % _BMC == 0 else (200 if n % 200 == 0 else n)
    bmb = _BMB if n % _BMB == 0 else n
    bf = jnp.bfloat16
    f32 = jnp.float32

    def dot(a, b):
        return jnp.dot(a, b, preferred_element_type=f32)

    def pad128(y):
        w = y.shape[1]
        if w == 128:
            return y
        return jnp.concatenate(
            [y, jnp.zeros((y.shape[0], 128 - w), f32)], axis=1)

    def proj_body(x_ref, h_ref):
        hv = jax.lax.dot_general(
            x_ref[...], wcat_ref[...],
            dimension_numbers=(((1,), (1,)), ((), ())),
            preferred_element_type=f32)
        h_ref[...] = pad128(hv)

    pltpu.emit_pipeline(
        proj_body, grid=(n // bmb,),
        in_specs=[pl.BlockSpec((bmb, 128), lambda i: (i, 0))],
        out_specs=[pl.BlockSpec((bmb, 128), lambda i: (i, 0))],
    )(x, x0)

    def prop_pass(a, xin, c0, c1, bm, xout=None, group=None, cache=None):
        def body(*refs):
            a_ref = refs[0]
            av = a_ref[...]
            if av.dtype == f32:
                abf = av.astype(bf)
                refs[-1][...] = abf
            else:
                abf = av
            y = pad128(dot(abf, xin[:, c0:c1].astype(bf)))
            k = 1
            if xout is not None:
                refs[k][...] = y
                k += 1
            if group is not None:
                refs[k][...] = y

        in_specs = [pl.BlockSpec((bm, n), lambda i: (i, 0))]
        out_specs = []
        refs = [a]
        if xout is not None:
            out_specs.append(pl.BlockSpec((bm, 128), lambda i: (i, 0)))
            refs.append(xout)
        if group is not None:
            out_specs.append(pl.BlockSpec((bm, 128), lambda i, g=group: (i, g)))
            refs.append(v)
        if cache is not None:
            out_specs.append(pl.BlockSpec((bm, n), lambda i: (i, 0)))
            refs.append(cache)
        pltpu.emit_pipeline(
            body, grid=(n // bm,),
            in_specs=in_specs, out_specs=out_specs,
        )(*refs)

    prop_pass(gcn, x0, 0, 48, bmc, xout=x1, group=0, cache=gbf)
    prop_pass(sct, x0, 48, 96, bmc, xout=x2, group=3, cache=sbf)
    prop_pass(gbf, x1, 16, 48, bmb, xout=x0, group=1)
    prop_pass(gbf, x0, 16, 32, bmb, group=2)
    prop_pass(sbf, x2, 0, 48, bmb, xout=x1, group=4)
    prop_pass(sbf, x1, 16, 48, bmb, xout=x2)
    prop_pass(sbf, x2, 0, 32, bmb, xout=x1, group=5)
    prop_pass(sbf, x1, 16, 32, bmb, xout=x2)
    prop_pass(sbf, x2, 0, 16, bmb, xout=x1)
    prop_pass(sbf, x1, 0, 16, bmb, xout=x2)

    def last_body(a_ref, g1_ref, g2_ref, g3_ref, s1_ref, s2_ref, s4_ref,
                  o_ref):
        s8 = dot(a_ref[...], x2[:, 0:16].astype(bf))
        b = bcat_ref[...]
        o_ref[:, 0:16] = jnp.maximum(g1_ref[:, 0:16] + b[:, 0:16], 0.0)
        o_ref[:, 16:32] = jnp.maximum(g2_ref[:, 0:16] + b[:, 16:32], 0.0)
        o_ref[:, 32:48] = jnp.maximum(g3_ref[:, 0:16] + b[:, 32:48], 0.0)
        o_ref[:, 48:64] = jnp.maximum(
            s1_ref[:, 0:16] - s2_ref[:, 0:16] + b[:, 48:64], 0.0)
        o_ref[:, 64:80] = jnp.maximum(
            s2_ref[:, 16:32] - s4_ref[:, 0:16] + b[:, 64:80], 0.0)
        o_ref[:, 80:96] = jnp.maximum(
            s4_ref[:, 16:32] - s8 + b[:, 80:96], 0.0)
        o_ref[:, 96:128] = jnp.zeros((o_ref.shape[0], 32), f32)

    in_specs = [pl.BlockSpec((bmb, n), lambda i: (i, 0))]
    in_specs += [pl.BlockSpec((bmb, 128), lambda i, g=g: (i, g))
                 for g in (0, 1, 2, 3, 4, 5)]
    pltpu.emit_pipeline(
        last_body, grid=(n // bmb,),
        in_specs=in_specs,
        out_specs=[pl.BlockSpec((bmb, 128), lambda i: (i, 6))],
    )(sbf, *([v] * 6), v)


def kernel(x, gcn_mat, sct_mat, W0, W1, W2, W3, W4, W5,
           b0, b1, b2, b3, b4, b5):
    n = x.shape[0]
    wcat = jnp.concatenate([W0, W1, W2, W3, W4, W5], axis=0)
    bcat = jnp.concatenate([b0, b1, b2, b3, b4, b5], axis=1)

    v, _, _ = pl.pallas_call(
        _mega_body,
        in_specs=[
            pl.BlockSpec(memory_space=_HBM),
            pl.BlockSpec(memory_space=_HBM),
            pl.BlockSpec(memory_space=_HBM),
            pl.BlockSpec(memory_space=_VMEM),
            pl.BlockSpec(memory_space=_VMEM),
        ],
        out_specs=[pl.BlockSpec(memory_space=_HBM)] * 3,
        out_shape=[
            jax.ShapeDtypeStruct((n, 7 * 128), jnp.float32),
            jax.ShapeDtypeStruct((n, n), jnp.bfloat16),
            jax.ShapeDtypeStruct((n, n), jnp.bfloat16),
        ],
        scratch_shapes=[
            pltpu.VMEM((n, 128), jnp.float32),
            pltpu.VMEM((n, 128), jnp.float32),
            pltpu.VMEM((n, 128), jnp.float32),
        ],
        compiler_params=pltpu.CompilerParams(
            vmem_limit_bytes=100 * 1024 * 1024,
        ),
    )(x, gcn_mat, sct_mat, wcat, bcat)
    return v[:, 6 * 128:6 * 128 + 96]
